# baseline (speedup 1.0000x reference)
import functools

import jax
import jax.numpy as jnp
from jax import lax
from jax.experimental import pallas as pl
from jax.experimental.pallas import tpu as pltpu

EPS = 1e-5


def _clamp(x, m):
    return jnp.clip(x, -m, m)


def _k1_body(h_ref, gw_ref, lt_ref):
    x = _clamp(h_ref[...], 1000.0)
    mean = jnp.mean(x, axis=1, keepdims=True)
    var = jnp.mean(jnp.square(x - mean), axis=1, keepdims=True)
    hn = (x - mean) * lax.rsqrt(var + 1e-5)
    hn = _clamp(hn, 100.0)
    lt = lax.dot_general(gw_ref[...], hn, (((1,), (1,)), ((), ())),
                         preferred_element_type=jnp.float32)
    lt_ref[...] = jnp.clip(lt, -20.0, 20.0)


def _router_logits(flat, gate_w, tb=512):
    n, h = flat.shape
    e = gate_w.shape[0]
    grid = n // tb
    return pl.pallas_call(
        _k1_body,
        grid=(grid,),
        in_specs=[
            pl.BlockSpec((tb, h), lambda i: (i, 0)),
            pl.BlockSpec((e, h), lambda i: (0, 0)),
        ],
        out_specs=pl.BlockSpec((e, tb), lambda i: (0, i)),
        out_shape=jax.ShapeDtypeStruct((e, n), jnp.float32),
    )(flat, gate_w)


def _k2_body(k, lt_ref, wn_ref, aux_ref):
    l = lt_ref[...]
    e, n = l.shape
    mx = jnp.max(l, axis=1, keepdims=True)
    ex = jnp.exp(l - mx)
    p = ex / jnp.sum(ex, axis=1, keepdims=True)
    p = jnp.clip(p, EPS, 1.0)
    u = lax.bitcast_convert_type(p, jnp.int32)

    def tstep(i, t):
        b = 30 - i
        cand = t | (1 << b)
        cnt = jnp.sum((u >= cand).astype(jnp.int32), axis=1, keepdims=True)
        return jnp.where(cnt >= k, cand, t)

    t = lax.fori_loop(0, 31, tstep, jnp.zeros((e, 1), jnp.int32))
    g = jnp.sum((u > t).astype(jnp.int32), axis=1, keepdims=True)
    need = k - g
    nidx = lax.broadcasted_iota(jnp.int32, (e, n), 1)
    tie = (u == t)

    def mstep(i, lo):
        b = 12 - i
        cand = lo + (1 << b)
        cnt = jnp.sum((tie & (nidx < cand)).astype(jnp.int32),
                      axis=1, keepdims=True)
        return jnp.where(cnt < need, cand, lo)

    lo = lax.fori_loop(0, 13, mstep, jnp.zeros((e, 1), jnp.int32))
    m = jnp.where(need > 0, lo + 1, 0)
    chosen = (u > t) | (tie & (nidx < m))

    counts = jnp.sum(jnp.where(chosen, p, 0.0), axis=0, keepdims=True)
    counts = jnp.maximum(counts, EPS)
    wn_ref[...] = jnp.where(chosen, p / counts, 0.0)

    mr = jnp.max(l, axis=0, keepdims=True)
    lse = jnp.log(jnp.sum(jnp.exp(l - mr), axis=0, keepdims=True)) + mr
    aux = jnp.mean(jnp.square(lse)) * 0.001
    aux_ref[...] = jnp.clip(aux, 0.0, 10.0)[None, None]


def _routing_weights(lt, k):
    e, n = lt.shape
    return pl.pallas_call(
        functools.partial(_k2_body, k),
        out_shape=(
            jax.ShapeDtypeStruct((e, n), jnp.float32),
            jax.ShapeDtypeStruct((1, 1), jnp.float32),
        ),
    )(lt)


def _k3_body(nib, h_ref, wg_ref, wu_ref, wd_ref, wn_ref, out_ref, acc_ref):
    ib = pl.program_id(2)
    ei = pl.program_id(1)
    x = _clamp(h_ref[...], 1000.0)
    g = lax.dot_general(x, wg_ref[0], (((1,), (1,)), ((), ())),
                        preferred_element_type=jnp.float32)
    g = _clamp(jax.nn.silu(g), 1000.0)
    u = lax.dot_general(x, wu_ref[0], (((1,), (1,)), ((), ())),
                        preferred_element_type=jnp.float32)
    u = _clamp(u, 1000.0)
    part = lax.dot_general(g * u, wd_ref[0], (((1,), (1,)), ((), ())),
                           preferred_element_type=jnp.float32)

    @pl.when(ib == 0)
    def _():
        acc_ref[...] = jnp.zeros_like(acc_ref)

    acc_ref[...] += part

    @pl.when(ib == nib - 1)
    def _():
        wn = wn_ref[...]
        lane = lax.broadcasted_iota(jnp.int32, wn.shape, 1)
        wcol = jnp.sum(jnp.where(lane == ei, wn, 0.0), axis=1, keepdims=True)
        o = _clamp(acc_ref[...], 1000.0) * wcol

        @pl.when(ei == 0)
        def _():
            out_ref[...] = jnp.zeros_like(out_ref)

        out_ref[...] += o

    ne = pl.num_programs(1)

    @pl.when((ib == nib - 1) & (ei == ne - 1))
    def _():
        out_ref[...] = _clamp(out_ref[...], 1000.0)


def _dense_moe(flat, w_gate, w_up, w_down, wn, tb=512, ibk=1024):
    n, h = flat.shape
    e, i, _ = w_gate.shape
    nib = i // ibk
    return pl.pallas_call(
        functools.partial(_k3_body, nib),
        grid=(n // tb, e, nib),
        in_specs=[
            pl.BlockSpec((tb, h), lambda ti, ei, ib: (ti, 0)),
            pl.BlockSpec((1, ibk, h), lambda ti, ei, ib: (ei, ib, 0)),
            pl.BlockSpec((1, ibk, h), lambda ti, ei, ib: (ei, ib, 0)),
            pl.BlockSpec((1, h, ibk), lambda ti, ei, ib: (ei, 0, ib)),
            pl.BlockSpec((tb, e), lambda ti, ei, ib: (ti, 0)),
        ],
        out_specs=pl.BlockSpec((tb, h), lambda ti, ei, ib: (ti, 0)),
        out_shape=jax.ShapeDtypeStruct((n, h), jnp.float32),
        scratch_shapes=[pltpu.VMEM((tb, h), jnp.float32)],
    )(flat, w_gate, w_up, w_down, wn)


def kernel(hidden_states, gate_w, w_gate, w_up, w_down):
    b, s, h = hidden_states.shape
    e = gate_w.shape[0]
    n = b * s
    capacity = max(int(n * 1.0 / e), 1)
    k = min(capacity, n)
    flat = hidden_states.reshape(n, h)
    lt = _router_logits(flat, gate_w)
    wn_t, aux = _routing_weights(lt, k)
    final = _dense_moe(flat, w_gate, w_up, w_down, wn_t.T)
    return final.reshape(b, s, h), aux[0, 0]
